# baseline (speedup 1.0000x reference)
import jax
import jax.numpy as jnp
from jax import lax
from jax.experimental import pallas as pl
from jax.experimental.pallas import tpu as pltpu
from jax.experimental.pallas import tpu_sc as plsc

N = 10000
E = 320000
NFEAT = 128
NHID = 64
NCLASS = 40
NPAD = 10240
GSZ = 128
EPG = 256
NGRP = E // EPG
NTILES = 32
RPT = NPAD // 16


def _tc_proj(x_ref, w_ref, asw_ref, adw_ref, h_ref, as_ref, ad_ref, gm_ref):
    h = lax.dot_general(x_ref[...], w_ref[...], (((1,), (1,)), ((), ())),
                        preferred_element_type=jnp.float32)
    h_ref[...] = jnp.concatenate(
        [h, jnp.zeros((NPAD - N, NHID), jnp.float32)], axis=0)
    a_s = jnp.sum(h * asw_ref[...], axis=1)
    a_d = jnp.sum(h * adw_ref[...], axis=1)
    zpad = jnp.zeros((NPAD - N,), jnp.float32)
    as_ref[...] = jnp.concatenate([a_s, zpad]).reshape(1, NPAD)
    ad_ref[...] = jnp.concatenate([a_d, zpad]).reshape(1, NPAD)
    gm_ref[...] = jnp.broadcast_to(jnp.max(a_s) + jnp.max(a_d), (1, 16))


def _sc_edge(edges_hbm, asrc_hbm, adst_hbm, gm_hbm, h_hbm,
             accO, denO, asrc_v, adst_v, den_v, gm_v,
             idx0, idx1, idx2, rows0, rows1, rows2,
             sidx0, sidx1, sidx2, ex0, ex1, ex2,
             acc_sh, semi0, semi1, semi2,
             semg0, semg1, semg2, sems0, sems1, sems2):
    c = lax.axis_index("c")
    s = lax.axis_index("s")
    w = s * 2 + c
    base = s * RPT
    pltpu.sync_copy(asrc_hbm.at[0], asrc_v)
    pltpu.sync_copy(adst_hbm.at[0], adst_v)
    pltpu.sync_copy(gm_hbm.at[0], gm_v)
    zv = jnp.zeros((16,), jnp.float32)

    def zero16(q, _):
        den_v[pl.ds(q * 16, 16)] = zv
        for t in range(NHID // 16):
            rows0[q % EPG, pl.ds(t * 16, 16)] = zv
        return 0

    lax.fori_loop(0, NPAD // 16, zero16, 0)
    for part in range(RPT // EPG):
        pltpu.sync_copy(rows0, acc_sh.at[pl.ds(base + part * EPG, EPG)])
    pltpu.sync_copy(rows0.at[pl.ds(0, RPT % EPG)],
                    acc_sh.at[pl.ds(base + (RPT // EPG) * EPG, RPT % EPG)])
    plsc.subcore_barrier()
    gmv = gm_v[...]
    nfull = NGRP // NTILES
    ng = jnp.where(w < NGRP % NTILES, nfull + 1, nfull)

    idx = (idx0, idx1, idx2)
    rows = (rows0, rows1, rows2)
    sidx = (sidx0, sidx1, sidx2)
    exb = (ex0, ex1, ex2)
    semi = (semi0, semi1, semi2)
    semg = (semg0, semg1, semg2)
    sems = (sems0, sems1, sems2)

    def start_idx(b, i):
        pltpu.async_copy(edges_hbm.at[:, w + i * NTILES], idx[b], semi[b])

    def wait_idx(b):
        pltpu.make_async_copy(edges_hbm.at[:, 0], idx[b], semi[b]).wait()

    def start_gather(b):
        for j in range(EPG // GSZ):
            pltpu.async_copy(h_hbm.at[idx[b].at[0, pl.ds(j * GSZ, GSZ)]],
                             rows[b].at[pl.ds(j * GSZ, GSZ)], semg[b])

    def wait_gather(b):
        for j in range(EPG // GSZ):
            pltpu.make_async_copy(h_hbm.at[idx[b].at[0, pl.ds(j * GSZ, GSZ)]],
                                  rows[b].at[pl.ds(j * GSZ, GSZ)],
                                  semg[b]).wait()

    def start_scatter(b):
        for j in range(EPG // GSZ):
            pltpu.async_copy(rows[b].at[pl.ds(j * GSZ, GSZ)],
                             acc_sh.at[sidx[b].at[j]], sems[b], add=True)

    def wait_scatter(b):
        for j in range(EPG // GSZ):
            pltpu.make_async_copy(rows[b].at[pl.ds(j * GSZ, GSZ)],
                                  acc_sh.at[sidx[b].at[j]], sems[b]).wait()

    start_idx(0, 0)
    start_idx(1, 1)
    start_idx(2, 2)
    wait_idx(0)
    start_gather(0)
    wait_idx(1)
    start_gather(1)

    def outer(o, _):
        for b in (0, 1, 2):
            i = 3 * o + b
            nb = (b + 2) % 3

            @pl.when(i + 2 < ng)
            def _():
                wait_idx(nb)

            @pl.when((i >= 1) & (i + 2 < ng))
            def _():
                wait_scatter(nb)

            @pl.when(i + 2 < ng)
            def _():
                start_gather(nb)

            @pl.when(i < ng)
            def _():
                for gi in range(EPG // 16):
                    sl = pl.ds(gi * 16, 16)
                    s16 = idx[b][0, sl]
                    d16 = idx[b][1, sl]
                    e = (plsc.load_gather(asrc_v, [s16])
                         + plsc.load_gather(adst_v, [d16]))
                    e = jnp.where(e > 0, e, 0.2 * e)
                    exv = jnp.exp(e - gmv)
                    exb[b][sl] = exv
                    sidx[b][gi // 8, pl.ds((gi % 8) * 16, 16)] = d16
                    plsc.addupdate_scatter(den_v, [d16], exv)
                wait_gather(b)

                def scale_chunk(q, _):
                    for t in range(4):
                        gi = q * 4 + t
                        exv = exb[b][pl.ds(gi * 16, 16)]
                        for k in range(16):
                            m = exv[k]
                            row = gi * 16 + k
                            for j in range(NHID // 16):
                                fsl = pl.ds(j * 16, 16)
                                rows[b][row, fsl] = rows[b][row, fsl] * m
                    return 0

                lax.fori_loop(0, EPG // 64, scale_chunk, 0)
                start_scatter(b)

            @pl.when(i + 3 < ng)
            def _():
                start_idx(b, i + 3)
        return 0

    lax.fori_loop(0, (nfull + 5) // 3, outer, 0)
    wait_scatter(0)
    wait_scatter(1)
    wait_scatter(2)
    pltpu.sync_copy(den_v, denO.at[c, s])
    plsc.subcore_barrier()
    pltpu.sync_copy(acc_sh.at[pl.ds(base, RPT)], accO.at[c, pl.ds(base, RPT)])


def _tc_epi(acc_ref, den_ref, bias_ref, gam_ref, bet_ref, fcw_ref, fcb_ref,
            out_ref):
    acc = acc_ref[0] + acc_ref[1]
    den = jnp.sum(den_ref[...], axis=(0, 1))
    o = acc / (den[:, None] + 1e-16)
    o = jnp.maximum(o + bias_ref[...], 0.0)
    o = o * (gam_ref[...] / jnp.sqrt(jnp.float32(1.0 + 1e-5))) + bet_ref[...]
    res = lax.dot_general(o, fcw_ref[...], (((1,), (1,)), ((), ())),
                          preferred_element_type=jnp.float32)
    out_ref[...] = (res + fcb_ref[...])[:N]


def kernel(edge_index, x, W, att_src, att_dst, bias, bn_gamma, bn_beta, fc_W,
           fc_b):
    h, asrc, adst, gm16 = pl.pallas_call(
        _tc_proj,
        out_shape=[
            jax.ShapeDtypeStruct((NPAD, NHID), jnp.float32),
            jax.ShapeDtypeStruct((1, NPAD), jnp.float32),
            jax.ShapeDtypeStruct((1, NPAD), jnp.float32),
            jax.ShapeDtypeStruct((1, 16), jnp.float32),
        ],
    )(x, W, att_src, att_dst)

    edges3d = edge_index.reshape(2, NGRP, EPG)

    sc = pl.kernel(
        _sc_edge,
        out_type=[
            jax.ShapeDtypeStruct((2, NPAD, NHID), jnp.float32),
            jax.ShapeDtypeStruct((2, 16, NPAD), jnp.float32),
        ],
        mesh=plsc.VectorSubcoreMesh(core_axis_name="c", subcore_axis_name="s"),
        compiler_params=pltpu.CompilerParams(
            needs_layout_passes=False, use_tc_tiling_on_sc=False),
        scratch_types=[
            pltpu.VMEM((NPAD,), jnp.float32),
            pltpu.VMEM((NPAD,), jnp.float32),
            pltpu.VMEM((NPAD,), jnp.float32),
            pltpu.VMEM((16,), jnp.float32),
        ] + [pltpu.VMEM((2, EPG), jnp.int32)] * 3
          + [pltpu.VMEM((EPG, NHID), jnp.float32)] * 3
          + [pltpu.VMEM((EPG // GSZ, GSZ), jnp.int32)] * 3
          + [pltpu.VMEM((EPG,), jnp.float32)] * 3
          + [pltpu.VMEM_SHARED((NPAD, NHID), jnp.float32)]
          + [pltpu.SemaphoreType.DMA] * 9,
    )
    acc2, den2 = sc(edges3d, asrc, adst, gm16, h)

    out = pl.pallas_call(
        _tc_epi,
        out_shape=jax.ShapeDtypeStruct((N, NCLASS), jnp.float32),
    )(acc2, den2, bias.reshape(1, NHID), bn_gamma.reshape(1, NHID),
      bn_beta.reshape(1, NHID), fc_W, fc_b.reshape(1, NCLASS))
    return out


# speedup vs baseline: 1.0459x; 1.0459x over previous
import jax
import jax.numpy as jnp
from jax import lax
from jax.experimental import pallas as pl
from jax.experimental.pallas import tpu as pltpu
from jax.experimental.pallas import tpu_sc as plsc

N = 10000
E = 320000
NFEAT = 128
NHID = 64
NCLASS = 40
NPAD = 10240
GSZ = 128
EPG = 256
NGRP = E // EPG
NTILES = 32
RPT = NPAD // 16


def _tc_proj(x_ref, w_ref, asw_ref, adw_ref, h_ref, as_ref, ad_ref, gm_ref):
    h = lax.dot_general(x_ref[...], w_ref[...], (((1,), (1,)), ((), ())),
                        preferred_element_type=jnp.float32)
    h_ref[...] = jnp.concatenate(
        [h, jnp.zeros((NPAD - N, NHID), jnp.float32)], axis=0)
    a_s = jnp.sum(h * asw_ref[...], axis=1)
    a_d = jnp.sum(h * adw_ref[...], axis=1)
    zpad = jnp.zeros((NPAD - N,), jnp.float32)
    as_ref[...] = jnp.concatenate([a_s, zpad]).reshape(1, NPAD)
    ad_ref[...] = jnp.concatenate([a_d, zpad]).reshape(1, NPAD)
    gm_ref[...] = jnp.broadcast_to(jnp.max(a_s) + jnp.max(a_d), (1, 16))


def _sc_edge(edges_hbm, asrc_hbm, adst_hbm, gm_hbm, h_hbm,
             accO, denO, asrc_v, adst_v, den_v, gm_v,
             idx0, idx1, rows0, rows1, sidx0, sidx1, ex0, ex1,
             acc_sh, semi0, semi1, semg0, semg1, sems0, sems1):
    c = lax.axis_index("c")
    s = lax.axis_index("s")
    w = s * 2 + c
    base = s * RPT
    pltpu.sync_copy(asrc_hbm.at[0], asrc_v)
    pltpu.sync_copy(adst_hbm.at[0], adst_v)
    pltpu.sync_copy(gm_hbm.at[0], gm_v)
    zv = jnp.zeros((16,), jnp.float32)

    def zero16(q, _):
        den_v[pl.ds(q * 16, 16)] = zv
        for t in range(NHID // 16):
            rows0[q % EPG, pl.ds(t * 16, 16)] = zv
        return 0

    lax.fori_loop(0, NPAD // 16, zero16, 0)
    for part in range(RPT // EPG):
        pltpu.sync_copy(rows0, acc_sh.at[pl.ds(base + part * EPG, EPG)])
    pltpu.sync_copy(rows0.at[pl.ds(0, RPT % EPG)],
                    acc_sh.at[pl.ds(base + (RPT // EPG) * EPG, RPT % EPG)])
    plsc.subcore_barrier()
    gmv = gm_v[...]
    nfull = NGRP // NTILES
    ng = jnp.where(w < NGRP % NTILES, nfull + 1, nfull)

    idx = (idx0, idx1)
    rows = (rows0, rows1)
    sidx = (sidx0, sidx1)
    exb = (ex0, ex1)
    semi = (semi0, semi1)
    semg = (semg0, semg1)
    sems = (sems0, sems1)

    def start_idx(b, i):
        pltpu.async_copy(edges_hbm.at[:, w + i * NTILES], idx[b], semi[b])

    def wait_idx(b):
        pltpu.make_async_copy(edges_hbm.at[:, 0], idx[b], semi[b]).wait()

    def start_gather(b):
        for j in range(EPG // GSZ):
            pltpu.async_copy(h_hbm.at[idx[b].at[0, pl.ds(j * GSZ, GSZ)]],
                             rows[b].at[pl.ds(j * GSZ, GSZ)], semg[b])

    def wait_gather(b):
        for j in range(EPG // GSZ):
            pltpu.make_async_copy(h_hbm.at[idx[b].at[0, pl.ds(j * GSZ, GSZ)]],
                                  rows[b].at[pl.ds(j * GSZ, GSZ)],
                                  semg[b]).wait()

    def start_scatter(b):
        for j in range(EPG // GSZ):
            pltpu.async_copy(rows[b].at[pl.ds(j * GSZ, GSZ)],
                             acc_sh.at[sidx[b].at[j]], sems[b], add=True)

    def wait_scatter(b):
        for j in range(EPG // GSZ):
            pltpu.make_async_copy(rows[b].at[pl.ds(j * GSZ, GSZ)],
                                  acc_sh.at[sidx[b].at[j]], sems[b]).wait()

    start_idx(0, 0)
    start_idx(1, 1)
    wait_idx(0)
    start_gather(0)

    def outer(o, _):
        for b in (0, 1):
            i = 2 * o + b
            nb = 1 - b

            @pl.when(i + 1 < ng)
            def _():
                wait_idx(nb)

            @pl.when((i >= 1) & (i + 1 < ng))
            def _():
                wait_scatter(nb)

            @pl.when(i + 1 < ng)
            def _():
                start_gather(nb)

            @pl.when(i < ng)
            def _():
                for gi in range(EPG // 16):
                    sl = pl.ds(gi * 16, 16)
                    s16 = idx[b][0, sl]
                    d16 = idx[b][1, sl]
                    e = (plsc.load_gather(asrc_v, [s16])
                         + plsc.load_gather(adst_v, [d16]))
                    e = jnp.where(e > 0, e, 0.2 * e)
                    exv = jnp.exp(e - gmv)
                    exb[b][sl] = exv
                    sidx[b][gi // 8, pl.ds((gi % 8) * 16, 16)] = d16
                    plsc.addupdate_scatter(den_v, [d16], exv)
                wait_gather(b)

                def scale_chunk(q, _):
                    for t in range(4):
                        gi = q * 4 + t
                        exv = exb[b][pl.ds(gi * 16, 16)]
                        for k in range(16):
                            m = exv[k]
                            row = gi * 16 + k
                            for j in range(NHID // 16):
                                fsl = pl.ds(j * 16, 16)
                                rows[b][row, fsl] = rows[b][row, fsl] * m
                    return 0

                lax.fori_loop(0, EPG // 64, scale_chunk, 0)
                start_scatter(b)

            @pl.when(i + 2 < ng)
            def _():
                start_idx(b, i + 2)
        return 0

    lax.fori_loop(0, (nfull + 3) // 2, outer, 0)
    wait_scatter(0)
    wait_scatter(1)
    pltpu.sync_copy(den_v, denO.at[c, s])
    plsc.subcore_barrier()
    pltpu.sync_copy(acc_sh.at[pl.ds(base, RPT)], accO.at[c, pl.ds(base, RPT)])


def _tc_epi(acc_ref, den_ref, bias_ref, gam_ref, bet_ref, fcw_ref, fcb_ref,
            out_ref):
    acc = acc_ref[0] + acc_ref[1]
    den = jnp.sum(den_ref[...], axis=(0, 1))
    o = acc / (den[:, None] + 1e-16)
    o = jnp.maximum(o + bias_ref[...], 0.0)
    o = o * (gam_ref[...] / jnp.sqrt(jnp.float32(1.0 + 1e-5))) + bet_ref[...]
    res = lax.dot_general(o, fcw_ref[...], (((1,), (1,)), ((), ())),
                          preferred_element_type=jnp.float32)
    out_ref[...] = (res + fcb_ref[...])[:N]


def kernel(edge_index, x, W, att_src, att_dst, bias, bn_gamma, bn_beta, fc_W,
           fc_b):
    h, asrc, adst, gm16 = pl.pallas_call(
        _tc_proj,
        out_shape=[
            jax.ShapeDtypeStruct((NPAD, NHID), jnp.float32),
            jax.ShapeDtypeStruct((1, NPAD), jnp.float32),
            jax.ShapeDtypeStruct((1, NPAD), jnp.float32),
            jax.ShapeDtypeStruct((1, 16), jnp.float32),
        ],
    )(x, W, att_src, att_dst)

    edges3d = edge_index.reshape(2, NGRP, EPG)

    sc = pl.kernel(
        _sc_edge,
        out_type=[
            jax.ShapeDtypeStruct((2, NPAD, NHID), jnp.float32),
            jax.ShapeDtypeStruct((2, 16, NPAD), jnp.float32),
        ],
        mesh=plsc.VectorSubcoreMesh(core_axis_name="c", subcore_axis_name="s"),
        compiler_params=pltpu.CompilerParams(
            needs_layout_passes=False, use_tc_tiling_on_sc=False),
        scratch_types=[
            pltpu.VMEM((NPAD,), jnp.float32),
            pltpu.VMEM((NPAD,), jnp.float32),
            pltpu.VMEM((NPAD,), jnp.float32),
            pltpu.VMEM((16,), jnp.float32),
        ] + [pltpu.VMEM((2, EPG), jnp.int32)] * 2
          + [pltpu.VMEM((EPG, NHID), jnp.float32)] * 2
          + [pltpu.VMEM((EPG // GSZ, GSZ), jnp.int32)] * 2
          + [pltpu.VMEM((EPG,), jnp.float32)] * 2
          + [pltpu.VMEM_SHARED((NPAD, NHID), jnp.float32)]
          + [pltpu.SemaphoreType.DMA] * 6,
    )
    acc2, den2 = sc(edges3d, asrc, adst, gm16, h)

    out = pl.pallas_call(
        _tc_epi,
        out_shape=jax.ShapeDtypeStruct((N, NCLASS), jnp.float32),
    )(acc2, den2, bias.reshape(1, NHID), bn_gamma.reshape(1, NHID),
      bn_beta.reshape(1, NHID), fc_W, fc_b.reshape(1, NCLASS))
    return out


# speedup vs baseline: 1.0571x; 1.0106x over previous
import jax
import jax.numpy as jnp
from jax import lax
from jax.experimental import pallas as pl
from jax.experimental.pallas import tpu as pltpu
from jax.experimental.pallas import tpu_sc as plsc

N = 10000
E = 320000
NFEAT = 128
NHID = 64
NCLASS = 40
NPAD = 10240
GSZ = 128
EPG = 256
NGRP = E // EPG
NTILES = 32
RPT = NPAD // 16


def _tc_proj(x_ref, w_ref, asw_ref, adw_ref, h_ref, ap_ref, gm_ref):
    w = w_ref[...]
    was = lax.dot_general(asw_ref[...], w, (((1,), (0,)), ((), ())),
                          preferred_element_type=jnp.float32)
    wad = lax.dot_general(adw_ref[...], w, (((1,), (0,)), ((), ())),
                          preferred_element_type=jnp.float32)
    w_aug = jnp.concatenate([w, was, wad], axis=0)
    ha = lax.dot_general(x_ref[...], w_aug, (((1,), (1,)), ((), ())),
                         preferred_element_type=jnp.float32)
    h_ref[...] = jnp.concatenate(
        [ha[:, :NHID], jnp.zeros((NPAD - N, NHID), jnp.float32)], axis=0)
    a2 = ha[:, NHID:NHID + 2]
    ap_ref[...] = jnp.concatenate(
        [a2, jnp.zeros((NPAD - N, 2), jnp.float32)], axis=0)
    gm_ref[...] = jnp.broadcast_to(
        jnp.max(a2[:, 0]) + jnp.max(a2[:, 1]), (1, 16))


def _sc_edge(edges_hbm, asrc_hbm, adst_hbm, gm_hbm, h_hbm,
             accO, denO, asrc_v, adst_v, den_v, gm_v,
             idx0, idx1, rows0, rows1, sidx0, sidx1, ex0, ex1,
             acc_sh, semi0, semi1, semg0, semg1, sems0, sems1):
    c = lax.axis_index("c")
    s = lax.axis_index("s")
    w = s * 2 + c
    base = s * RPT
    pltpu.sync_copy(asrc_hbm, asrc_v)
    pltpu.sync_copy(adst_hbm, adst_v)
    pltpu.sync_copy(gm_hbm.at[0], gm_v)
    zv = jnp.zeros((16,), jnp.float32)

    def zero16(q, _):
        den_v[pl.ds(q * 16, 16)] = zv
        for t in range(NHID // 16):
            rows0[q % EPG, pl.ds(t * 16, 16)] = zv
        return 0

    lax.fori_loop(0, NPAD // 16, zero16, 0)
    for part in range(RPT // EPG):
        pltpu.sync_copy(rows0, acc_sh.at[pl.ds(base + part * EPG, EPG)])
    pltpu.sync_copy(rows0.at[pl.ds(0, RPT % EPG)],
                    acc_sh.at[pl.ds(base + (RPT // EPG) * EPG, RPT % EPG)])
    plsc.subcore_barrier()
    gmv = gm_v[...]
    nfull = NGRP // NTILES
    ng = jnp.where(w < NGRP % NTILES, nfull + 1, nfull)

    idx = (idx0, idx1)
    rows = (rows0, rows1)
    sidx = (sidx0, sidx1)
    exb = (ex0, ex1)
    semi = (semi0, semi1)
    semg = (semg0, semg1)
    sems = (sems0, sems1)

    def start_idx(b, i):
        pltpu.async_copy(edges_hbm.at[:, w + i * NTILES], idx[b], semi[b])

    def wait_idx(b):
        pltpu.make_async_copy(edges_hbm.at[:, 0], idx[b], semi[b]).wait()

    def start_gather(b):
        for j in range(EPG // GSZ):
            pltpu.async_copy(h_hbm.at[idx[b].at[0, pl.ds(j * GSZ, GSZ)]],
                             rows[b].at[pl.ds(j * GSZ, GSZ)], semg[b])

    def wait_gather(b):
        for j in range(EPG // GSZ):
            pltpu.make_async_copy(h_hbm.at[idx[b].at[0, pl.ds(j * GSZ, GSZ)]],
                                  rows[b].at[pl.ds(j * GSZ, GSZ)],
                                  semg[b]).wait()

    def start_scatter(b):
        for j in range(EPG // GSZ):
            pltpu.async_copy(rows[b].at[pl.ds(j * GSZ, GSZ)],
                             acc_sh.at[sidx[b].at[j]], sems[b], add=True)

    def wait_scatter(b):
        for j in range(EPG // GSZ):
            pltpu.make_async_copy(rows[b].at[pl.ds(j * GSZ, GSZ)],
                                  acc_sh.at[sidx[b].at[j]], sems[b]).wait()

    start_idx(0, 0)
    start_idx(1, 1)
    wait_idx(0)
    start_gather(0)

    def outer(o, _):
        for b in (0, 1):
            i = 2 * o + b
            nb = 1 - b

            @pl.when(i + 1 < ng)
            def _():
                wait_idx(nb)

            @pl.when((i >= 1) & (i + 1 < ng))
            def _():
                wait_scatter(nb)

            @pl.when(i + 1 < ng)
            def _():
                start_gather(nb)

            @pl.when(i < ng)
            def _():
                for gi in range(EPG // 16):
                    sl = pl.ds(gi * 16, 16)
                    s16 = idx[b][0, sl]
                    d16 = idx[b][1, sl]
                    e = (plsc.load_gather(asrc_v, [s16])
                         + plsc.load_gather(adst_v, [d16]))
                    e = jnp.where(e > 0, e, 0.2 * e)
                    exv = jnp.exp(e - gmv)
                    exb[b][sl] = exv
                    sidx[b][gi // 8, pl.ds((gi % 8) * 16, 16)] = d16
                    plsc.addupdate_scatter(den_v, [d16], exv)
                wait_gather(b)

                def scale_chunk(q, _):
                    for t in range(4):
                        gi = q * 4 + t
                        exv = exb[b][pl.ds(gi * 16, 16)]
                        for k in range(16):
                            m = exv[k]
                            row = gi * 16 + k
                            for j in range(NHID // 16):
                                fsl = pl.ds(j * 16, 16)
                                rows[b][row, fsl] = rows[b][row, fsl] * m
                    return 0

                lax.fori_loop(0, EPG // 64, scale_chunk, 0)
                start_scatter(b)

            @pl.when(i + 2 < ng)
            def _():
                start_idx(b, i + 2)
        return 0

    lax.fori_loop(0, (nfull + 3) // 2, outer, 0)
    wait_scatter(0)
    wait_scatter(1)
    pltpu.sync_copy(den_v, denO.at[c, s])
    plsc.subcore_barrier()
    pltpu.sync_copy(acc_sh.at[pl.ds(base, RPT)], accO.at[c, pl.ds(base, RPT)])


def _tc_epi(acc_ref, den_ref, bias_ref, gam_ref, bet_ref, fcw_ref, fcb_ref,
            out_ref):
    acc = acc_ref[0] + acc_ref[1]
    den = jnp.sum(den_ref[...], axis=(0, 1))
    o = acc / (den[:, None] + 1e-16)
    o = jnp.maximum(o + bias_ref[...], 0.0)
    o = o * (gam_ref[...] / jnp.sqrt(jnp.float32(1.0 + 1e-5))) + bet_ref[...]
    res = lax.dot_general(o, fcw_ref[...], (((1,), (1,)), ((), ())),
                          preferred_element_type=jnp.float32)
    out_ref[...] = (res + fcb_ref[...])[:N]


def kernel(edge_index, x, W, att_src, att_dst, bias, bn_gamma, bn_beta, fc_W,
           fc_b):
    h, ap, gm16 = pl.pallas_call(
        _tc_proj,
        out_shape=[
            jax.ShapeDtypeStruct((NPAD, NHID), jnp.float32),
            jax.ShapeDtypeStruct((NPAD, 2), jnp.float32),
            jax.ShapeDtypeStruct((1, 16), jnp.float32),
        ],
    )(x, W, att_src, att_dst)

    edges3d = edge_index.reshape(2, NGRP, EPG)

    sc = pl.kernel(
        _sc_edge,
        out_type=[
            jax.ShapeDtypeStruct((2, NPAD, NHID), jnp.float32),
            jax.ShapeDtypeStruct((2, 16, NPAD), jnp.float32),
        ],
        mesh=plsc.VectorSubcoreMesh(core_axis_name="c", subcore_axis_name="s"),
        compiler_params=pltpu.CompilerParams(
            needs_layout_passes=False, use_tc_tiling_on_sc=False),
        scratch_types=[
            pltpu.VMEM((NPAD,), jnp.float32),
            pltpu.VMEM((NPAD,), jnp.float32),
            pltpu.VMEM((NPAD,), jnp.float32),
            pltpu.VMEM((16,), jnp.float32),
        ] + [pltpu.VMEM((2, EPG), jnp.int32)] * 2
          + [pltpu.VMEM((EPG, NHID), jnp.float32)] * 2
          + [pltpu.VMEM((EPG // GSZ, GSZ), jnp.int32)] * 2
          + [pltpu.VMEM((EPG,), jnp.float32)] * 2
          + [pltpu.VMEM_SHARED((NPAD, NHID), jnp.float32)]
          + [pltpu.SemaphoreType.DMA] * 6,
    )
    acc2, den2 = sc(edges3d, ap[:, 0], ap[:, 1], gm16, h)

    out = pl.pallas_call(
        _tc_epi,
        out_shape=jax.ShapeDtypeStruct((N, NCLASS), jnp.float32),
    )(acc2, den2, bias.reshape(1, NHID), bn_gamma.reshape(1, NHID),
      bn_beta.reshape(1, NHID), fc_W, fc_b.reshape(1, NCLASS))
    return out
